# baseline (speedup 1.0000x reference)
import functools
import math

import jax
import jax.numpy as jnp
from jax import lax
from jax.experimental import pallas as pl
from jax.experimental.pallas import tpu as pltpu
from jax.experimental.pallas import tpu_sc as plsc

D_MODEL = 1024
FPB = 43
BPB = 4
BPP = 4
DPS = D_MODEL // 4
PERIOD = FPB * BPB * BPP
PAT = 768
BS = PERIOD

_info = plsc.get_sparse_core_info()
NW = _info.num_cores * _info.num_subcores
RPW = PAT // NW


@functools.partial(
    pl.kernel,
    mesh=plsc.VectorSubcoreMesh(core_axis_name="c", subcore_axis_name="s"),
    out_type=jax.ShapeDtypeStruct((3 * PAT, DPS), jnp.float32),
    scratch_types=[
        pltpu.VMEM((FPB + BPB + BPP, DPS), jnp.float32),
        pltpu.VMEM((3 * RPW, DPS), jnp.float32),
        pltpu.SemaphoreType.DMA,
    ],
)
def _gather_pattern(tab_hbm, out_hbm, tab_v, rows_v, wsem):
    wid = lax.axis_index("s") * _info.num_cores + lax.axis_index("c")
    base = wid * (3 * RPW)
    pltpu.sync_copy(tab_hbm, tab_v)

    def _copy_rows(i, carry):
        for u in range(4):
            q = i * 4 + u
            sq = base + q
            part = sq // PAT
            r = sq % PAT
            rowf = r % FPB
            rowb = FPB + (r // FPB) % BPB
            rowp = FPB + BPB + (r // (FPB * BPB)) % BPP
            row = jnp.where(part == 0, rowf, jnp.where(part == 1, rowb, rowp))
            for c in range(DPS // 16):
                rows_v[q, pl.ds(c * 16, 16)] = tab_v[row, pl.ds(c * 16, 16)]
        return carry

    lax.fori_loop(0, 3 * RPW // 4, _copy_rows, 0)
    pltpu.async_copy(rows_v, out_hbm.at[pl.ds(base, 3 * RPW)], wsem).wait()


def _add_pe_kernel(pf_ref, pb_ref, pp_ref, fp_ref, x_ref, o_ref):
    j = pl.program_id(0)
    row = j * BS + jax.lax.broadcasted_iota(jnp.int32, (BS, 1), 0)
    freq = fp_ref[0:1, :]
    phase = fp_ref[1:2, :]
    abs_pe = jnp.sin(row.astype(jnp.float32) * freq + phase)
    enc = jnp.concatenate(
        [pf_ref[:BS], pb_ref[:BS], pp_ref[:BS], abs_pe], axis=-1)
    o_ref[...] = x_ref[...] + enc[None, :, :]


def kernel(x, frame_embed, beat_embed, bar_embed, pe):
    B, S, D = x.shape
    table = jnp.concatenate([frame_embed, beat_embed, bar_embed], axis=0)
    pat = _gather_pattern(table)
    lane = jnp.arange(DPS)
    freq = jnp.exp((lane // 2 * 2).astype(jnp.float32) * (-math.log(10000.0) / DPS))
    phase = jnp.where(lane % 2 == 1, jnp.float32(math.pi / 2), jnp.float32(0.0))
    fp = jnp.zeros((8, DPS), x.dtype).at[0].set(freq).at[1].set(phase)
    return pl.pallas_call(
        _add_pe_kernel,
        grid=(pl.cdiv(S, BS),),
        in_specs=[
            pl.BlockSpec((PAT, DPS), lambda j: (0, 0)),
            pl.BlockSpec((PAT, DPS), lambda j: (1, 0)),
            pl.BlockSpec((PAT, DPS), lambda j: (2, 0)),
            pl.BlockSpec((8, DPS), lambda j: (0, 0)),
            pl.BlockSpec((B, BS, D), lambda j: (0, j, 0)),
        ],
        out_specs=pl.BlockSpec((B, BS, D), lambda j: (0, j, 0)),
        out_shape=jax.ShapeDtypeStruct((B, S, D), x.dtype),
        compiler_params=pltpu.CompilerParams(
            dimension_semantics=("parallel",),
        ),
    )(pat, pat, pat, fp, x)


# speedup vs baseline: 1.0053x; 1.0053x over previous
import functools
import math

import jax
import jax.numpy as jnp
from jax import lax
from jax.experimental import pallas as pl
from jax.experimental.pallas import tpu as pltpu
from jax.experimental.pallas import tpu_sc as plsc

D_MODEL = 1024
FPB = 43
BPB = 4
BPP = 4
DPS = D_MODEL // 4
PERIOD = FPB * BPB * BPP
PAT = 768
BS = PERIOD

_info = plsc.get_sparse_core_info()
NW = _info.num_cores * _info.num_subcores
RPW = PAT // NW


@functools.partial(
    pl.kernel,
    mesh=plsc.VectorSubcoreMesh(core_axis_name="c", subcore_axis_name="s"),
    out_type=jax.ShapeDtypeStruct((3 * PAT, DPS), jnp.float32),
    scratch_types=[
        pltpu.VMEM((FPB + BPB + BPP, DPS), jnp.float32),
        pltpu.VMEM((3 * RPW, DPS), jnp.float32),
        pltpu.SemaphoreType.DMA,
    ],
)
def _gather_pattern(tab_hbm, out_hbm, tab_v, rows_v, wsem):
    wid = lax.axis_index("s") * _info.num_cores + lax.axis_index("c")
    base = wid * (3 * RPW)
    pltpu.sync_copy(tab_hbm, tab_v)

    def _copy_row(q, carry):
        sq = base + q
        part = sq // PAT
        r = sq % PAT
        rowf = r % FPB
        rowb = FPB + (r // FPB) % BPB
        rowp = FPB + BPB + (r // (FPB * BPB)) % BPP
        row = jnp.where(part == 0, rowf, jnp.where(part == 1, rowb, rowp))
        for c in range(DPS // 16):
            rows_v[q, pl.ds(c * 16, 16)] = tab_v[row, pl.ds(c * 16, 16)]
        return carry

    lax.fori_loop(0, 3 * RPW, _copy_row, 0)
    pltpu.async_copy(rows_v, out_hbm.at[pl.ds(base, 3 * RPW)], wsem).wait()


def _add_pe_kernel(pf_ref, pb_ref, pp_ref, fp_ref, x_ref, o_ref):
    j = pl.program_id(0)
    row = j * BS + jax.lax.broadcasted_iota(jnp.int32, (BS, 1), 0)
    freq = fp_ref[0:1, :]
    phase = fp_ref[1:2, :]
    abs_pe = jnp.sin(row.astype(jnp.float32) * freq + phase)
    enc = jnp.concatenate(
        [pf_ref[:BS], pb_ref[:BS], pp_ref[:BS], abs_pe], axis=-1)
    o_ref[...] = x_ref[...] + enc[None, :, :]


def kernel(x, frame_embed, beat_embed, bar_embed, pe):
    B, S, D = x.shape
    table = jnp.concatenate([frame_embed, beat_embed, bar_embed], axis=0)
    pat = _gather_pattern(table)
    lane = jnp.arange(DPS)
    freq = jnp.exp((lane // 2 * 2).astype(jnp.float32) * (-math.log(10000.0) / DPS))
    phase = jnp.where(lane % 2 == 1, jnp.float32(math.pi / 2), jnp.float32(0.0))
    fp = jnp.zeros((8, DPS), x.dtype).at[0].set(freq).at[1].set(phase)
    return pl.pallas_call(
        _add_pe_kernel,
        grid=(pl.cdiv(S, BS),),
        in_specs=[
            pl.BlockSpec((PAT, DPS), lambda j: (0, 0)),
            pl.BlockSpec((PAT, DPS), lambda j: (1, 0)),
            pl.BlockSpec((PAT, DPS), lambda j: (2, 0)),
            pl.BlockSpec((8, DPS), lambda j: (0, 0)),
            pl.BlockSpec((B, BS, D), lambda j: (0, j, 0)),
        ],
        out_specs=pl.BlockSpec((B, BS, D), lambda j: (0, j, 0)),
        out_shape=jax.ShapeDtypeStruct((B, S, D), x.dtype),
        compiler_params=pltpu.CompilerParams(
            dimension_semantics=("parallel",),
        ),
    )(pat, pat, pat, fp, x)
